# baseline (speedup 1.0000x reference)
import functools
import math

import jax
import jax.numpy as jnp
from jax import lax
from jax.experimental import pallas as pl
from jax.experimental.pallas import tpu as pltpu
from jax.experimental.pallas import tpu_sc as plsc

D_MODEL = 64
SCALE = math.sqrt(D_MODEL)
NBUF = 3
LA = 2
BLK = 128


@functools.lru_cache(maxsize=None)
def _build(n_i0, n_i1, D):
    NC, NS = 2, 16
    NW = NC * NS
    n_b0 = n_i0 // BLK
    n_blocks = n_i1 * n_b0
    assert n_blocks % NW == 0
    blocks_per_w = n_blocks // NW
    GD = D // 8
    TAIL = NBUF + (blocks_per_w - 2 * NBUF) % NBUF
    MAIN_END = blocks_per_w - TAIL
    assert blocks_per_w > NBUF + TAIL and (MAIN_END - NBUF) % NBUF == 0

    mesh = plsc.VectorSubcoreMesh(
        core_axis_name="c", subcore_axis_name="s", num_cores=NC, num_subcores=NS
    )

    @functools.partial(
        pl.kernel,
        mesh=mesh,
        out_type=jax.ShapeDtypeStruct((n_i1 * GD * n_b0 * 8 * BLK,), jnp.float32),
        scratch_types=[
            pltpu.VMEM((blocks_per_w, BLK), jnp.int32),
            pltpu.VMEM((NBUF, BLK, D), jnp.float32),
            pltpu.VMEM((NBUF, D * BLK), jnp.float32),
            pltpu.SemaphoreType.DMA((NBUF,)),
            pltpu.SemaphoreType.DMA((NBUF,)),
        ],
        compiler_params=pltpu.CompilerParams(
            use_tc_tiling_on_sc=False, needs_layout_passes=False
        ),
    )
    def emb_kernel(idx_hbm, table_hbm, out_hbm, idx_v, gbuf, sbuf, gsem, ssem):
        wid = lax.axis_index("s") * NC + lax.axis_index("c")
        base_b = wid * blocks_per_w
        pltpu.sync_copy(idx_hbm.at[pl.ds(base_b, blocks_per_w)], idx_v)
        l_iota = lax.iota(jnp.int32, 16)

        def gather_start(j, b):
            pltpu.async_copy(table_hbm.at[idx_v.at[j]], gbuf.at[b], gsem.at[b])

        def gather_wait(b):
            pltpu.make_async_copy(
                table_hbm.at[idx_v.at[0]], gbuf.at[b], gsem.at[b]
            ).wait()

        def transpose_scale(b):
            @plsc.parallel_loop(0, D, unroll=4)
            def _diag(c):
                f = jnp.bitwise_and(c + l_iota, D - 1)
                sidx0 = (
                    jnp.right_shift(f, 3) * (8 * BLK)
                    + jnp.bitwise_and(f, 7) * BLK
                )
                for l0 in range(0, BLK, 16):
                    row = l0 + l_iota
                    v = plsc.load_gather(gbuf.at[b], [row, f])
                    plsc.store_scatter(sbuf.at[b], [sidx0 + row], v * SCALE)

        def store_start(j, b):
            B = base_b + j
            i1 = B // n_b0
            b0 = B % n_b0
            base = (i1 * GD * n_b0 + b0) * 8 * BLK
            for g in range(GD):
                pltpu.async_copy(
                    sbuf.at[b, pl.ds(g * 8 * BLK, 8 * BLK)],
                    out_hbm.at[pl.ds(base + g * n_b0 * 8 * BLK, 8 * BLK)],
                    ssem.at[b],
                )

        def store_wait(b):
            pltpu.make_async_copy(
                table_hbm.at[idx_v.at[0]], gbuf.at[b], ssem.at[b]
            ).wait()

        for j in range(LA):
            gather_start(j, j)

        def step(j, b, *, wait_store, issue):
            b2 = (j + LA) % NBUF
            if wait_store:
                store_wait(b2)
            if issue:
                gather_start(j + LA, b2)
            gather_wait(b)
            transpose_scale(b)
            store_start(j, b)

        for j in range(NBUF):
            step(j, j % NBUF, wait_store=(j >= NBUF - LA), issue=True)

        @pl.loop(NBUF, MAIN_END, step=NBUF)
        def _main(j0):
            for k in range(NBUF):
                step(j0 + k, k, wait_store=True, issue=True)

        for k in range(TAIL):
            j = MAIN_END + k
            step(j, j % NBUF, wait_store=True, issue=(j + LA < blocks_per_w))

        store_wait((blocks_per_w - 1) % NBUF)

    return emb_kernel


def kernel(x, table):
    n_i0, n_i1 = x.shape
    D = table.shape[1]
    n_b0 = n_i0 // BLK
    idx = x.T.reshape(n_i1 * n_b0, BLK).astype(jnp.int32)
    out = _build(n_i0, n_i1, D)(idx, table)
    out = out.reshape(n_i1, D // 8, n_b0, 8, BLK)
    out = out.transpose(2, 4, 0, 1, 3)
    return out.reshape(n_i0, n_i1, D)


# speedup vs baseline: 1.0116x; 1.0116x over previous
import functools
import math

import jax
import jax.numpy as jnp
from jax import lax
from jax.experimental import pallas as pl
from jax.experimental.pallas import tpu as pltpu
from jax.experimental.pallas import tpu_sc as plsc

D_MODEL = 64
SCALE = math.sqrt(D_MODEL)
NBUF = 3
LA = 2
BLK = 128


@functools.lru_cache(maxsize=None)
def _build(n_i0, n_i1, D):
    NC, NS = 2, 16
    NW = NC * NS
    n_b0 = n_i0 // BLK
    n_blocks = n_i1 * n_b0
    assert n_blocks % NW == 0
    blocks_per_w = n_blocks // NW
    GD = D // 8
    TAIL = NBUF + (blocks_per_w - 2 * NBUF) % NBUF
    MAIN_END = blocks_per_w - TAIL
    assert blocks_per_w > NBUF + TAIL and (MAIN_END - NBUF) % NBUF == 0

    mesh = plsc.VectorSubcoreMesh(
        core_axis_name="c", subcore_axis_name="s", num_cores=NC, num_subcores=NS
    )

    @functools.partial(
        pl.kernel,
        mesh=mesh,
        out_type=jax.ShapeDtypeStruct((n_i1 * GD * n_b0 * 8 * BLK,), jnp.float32),
        scratch_types=[
            pltpu.VMEM((blocks_per_w, BLK), jnp.int32),
            pltpu.VMEM((NBUF, BLK, D), jnp.float32),
            pltpu.VMEM((NBUF, D * BLK), jnp.float32),
            pltpu.SemaphoreType.DMA((NBUF,)),
            pltpu.SemaphoreType.DMA((NBUF,)),
        ],
        compiler_params=pltpu.CompilerParams(
            use_tc_tiling_on_sc=False, needs_layout_passes=False
        ),
    )
    def emb_kernel(idx_hbm, table_hbm, out_hbm, idx_v, gbuf, sbuf, gsem, ssem):
        wid = lax.axis_index("s") * NC + lax.axis_index("c")
        base_b = wid * blocks_per_w
        pltpu.sync_copy(idx_hbm.at[pl.ds(base_b, blocks_per_w)], idx_v)
        l_iota = lax.iota(jnp.int32, 16)

        def gather_start(j, b):
            pltpu.async_copy(table_hbm.at[idx_v.at[j]], gbuf.at[b], gsem.at[b])

        def gather_wait(b):
            pltpu.make_async_copy(
                table_hbm.at[idx_v.at[0]], gbuf.at[b], gsem.at[b]
            ).wait()

        def transpose_scale(b):
            @plsc.parallel_loop(0, D, unroll=2)
            def _diag(c):
                f = jnp.bitwise_and(c + l_iota, D - 1)
                sidx0 = (
                    jnp.right_shift(f, 3) * (8 * BLK)
                    + jnp.bitwise_and(f, 7) * BLK
                )
                for l0 in range(0, BLK, 16):
                    row = l0 + l_iota
                    v = plsc.load_gather(gbuf.at[b], [row, f])
                    plsc.store_scatter(sbuf.at[b], [sidx0 + row], v * SCALE)

        def store_start(j, b):
            B = base_b + j
            i1 = B // n_b0
            b0 = B % n_b0
            base = (i1 * GD * n_b0 + b0) * 8 * BLK
            for g in range(GD):
                pltpu.async_copy(
                    sbuf.at[b, pl.ds(g * 8 * BLK, 8 * BLK)],
                    out_hbm.at[pl.ds(base + g * n_b0 * 8 * BLK, 8 * BLK)],
                    ssem.at[b],
                )

        def store_wait(b):
            pltpu.make_async_copy(
                table_hbm.at[idx_v.at[0]], gbuf.at[b], ssem.at[b]
            ).wait()

        for j in range(LA):
            gather_start(j, j)

        def step(j, b, *, wait_store, issue):
            b2 = (j + LA) % NBUF
            if wait_store:
                store_wait(b2)
            if issue:
                gather_start(j + LA, b2)
            gather_wait(b)
            transpose_scale(b)
            store_start(j, b)

        for j in range(NBUF):
            step(j, j % NBUF, wait_store=(j >= NBUF - LA), issue=True)

        @pl.loop(NBUF, MAIN_END, step=NBUF)
        def _main(j0):
            for k in range(NBUF):
                step(j0 + k, k, wait_store=True, issue=True)

        for k in range(TAIL):
            j = MAIN_END + k
            step(j, j % NBUF, wait_store=True, issue=(j + LA < blocks_per_w))

        store_wait((blocks_per_w - 1) % NBUF)

    return emb_kernel


def kernel(x, table):
    n_i0, n_i1 = x.shape
    D = table.shape[1]
    n_b0 = n_i0 // BLK
    idx = x.T.reshape(n_i1 * n_b0, BLK).astype(jnp.int32)
    out = _build(n_i0, n_i1, D)(idx, table)
    out = out.reshape(n_i1, D // 8, n_b0, 8, BLK)
    out = out.transpose(2, 4, 0, 1, 3)
    return out.reshape(n_i0, n_i1, D)
